# Initial kernel scaffold; baseline (speedup 1.0000x reference)
import functools

import jax
import jax.numpy as jnp
from jax import lax
from jax.experimental import pallas as pl
from jax.experimental.pallas import tpu as pltpu
from jax.experimental.pallas import tpu_sc as plsc

N = 10000
F_IN = 128
HEADS = 8
PER_HEAD = 16
N_CLASSES = 16
HIDDEN = HEADS * PER_HEAD

N_PAD = 10240
E_RAW = 320000 + N
NTILES = 32
CHUNK = 128
EPT = 10368
E_PAD = NTILES * EPT
BT = 256


def _sc_edge_body(width, heads, nsc,
                  src_r, dst_r, hx_r, ad_r, out_r,
                  sidx, didx, hbuf, adbuf, acc, sem1, sem2):
    hoff = heads * PER_HEAD
    c = lax.axis_index("c")
    s = lax.axis_index("s")
    wid = s * nsc + c
    nvec = width // 16
    rpt = N_PAD // 16

    def zrow(e, _):
        for v in range(nvec):
            hbuf[e, pl.ds(16 * v, 16)] = jnp.zeros((16,), jnp.float32)
        return 0
    lax.fori_loop(0, CHUNK, zrow, 0)
    for k in range(rpt // CHUNK):
        pltpu.sync_copy(hbuf, acc.at[pl.ds(s * rpt + k * CHUNK, CHUNK)])
    plsc.subcore_barrier()

    lane = lax.iota(jnp.int32, 16)
    lmask = lane < heads

    def chunk_body(i, _):
        base = wid * EPT + i * CHUNK
        pltpu.sync_copy(src_r.at[pl.ds(base, CHUNK)], sidx)
        pltpu.sync_copy(dst_r.at[pl.ds(base, CHUNK)], didx)
        cp1 = pltpu.async_copy(hx_r.at[sidx], hbuf, sem1)
        cp2 = pltpu.async_copy(ad_r.at[didx], adbuf, sem2)
        cp1.wait()
        cp2.wait()

        def edge(e, _):
            asv = hbuf[e, pl.ds(hoff, 16)]
            adv = adbuf[e, :]
            z = asv + adv
            lr = jnp.maximum(z, 0.2 * z)
            ev = jnp.where(lmask, jnp.exp(lr), 0.0)
            hbuf[e, pl.ds(hoff, 16)] = ev
            for j in range(heads):
                sc = hbuf[e, hoff + j]
                hv = hbuf[e, pl.ds(16 * j, 16)]
                hbuf[e, pl.ds(16 * j, 16)] = sc * hv
            return 0
        lax.fori_loop(0, CHUNK, edge, 0)

        pltpu.sync_copy(hbuf, acc.at[didx], add=True)
        return 0
    lax.fori_loop(0, EPT // CHUNK, chunk_body, 0)

    plsc.subcore_barrier()
    pltpu.sync_copy(acc.at[pl.ds(s * rpt, rpt)],
                    out_r.at[c].at[pl.ds(s * rpt, rpt)])


def _make_sc_edge(width, heads):
    info = plsc.get_sparse_core_info()
    nsc = info.num_cores
    mesh = plsc.VectorSubcoreMesh(core_axis_name="c", subcore_axis_name="s")
    return functools.partial(
        pl.kernel,
        out_type=jax.ShapeDtypeStruct((nsc, N_PAD, width), jnp.float32),
        mesh=mesh,
        scratch_types=[
            pltpu.VMEM((CHUNK,), jnp.int32),
            pltpu.VMEM((CHUNK,), jnp.int32),
            pltpu.VMEM((CHUNK, width), jnp.float32),
            pltpu.VMEM((CHUNK, 16), jnp.float32),
            pltpu.VMEM_SHARED((N_PAD, width), jnp.float32),
            pltpu.SemaphoreType.DMA,
            pltpu.SemaphoreType.DMA,
        ],
    )(functools.partial(_sc_edge_body, width, heads, nsc))


def _prep_body(x_r, we_r, wa_r, he_r, ad_r):
    x = x_r[...]
    he_r[...] = jnp.dot(x, we_r[...], preferred_element_type=jnp.float32)
    ad_r[...] = jnp.dot(x, wa_r[...], preferred_element_type=jnp.float32)


def _prep(xp, wext, wad):
    width = wext.shape[1]
    return pl.pallas_call(
        _prep_body,
        grid=(N_PAD // BT,),
        in_specs=[
            pl.BlockSpec((BT, F_IN), lambda i: (i, 0)),
            pl.BlockSpec((F_IN, width), lambda i: (0, 0)),
            pl.BlockSpec((F_IN, 16), lambda i: (0, 0)),
        ],
        out_specs=[
            pl.BlockSpec((BT, width), lambda i: (i, 0)),
            pl.BlockSpec((BT, 16), lambda i: (i, 0)),
        ],
        out_shape=[
            jax.ShapeDtypeStruct((N_PAD, width), jnp.float32),
            jax.ShapeDtypeStruct((N_PAD, 16), jnp.float32),
        ],
    )(xp, wext, wad)


def _combine_prep_body(parts_r, b_r, p8_r, we_r, wa_r, he_r, ad_r):
    p = parts_r[0] + parts_r[1]
    h = p[:, :HIDDEN]
    den = p[:, HIDDEN:HIDDEN + HEADS]
    recip = 1.0 / (den + 1e-16)
    rep = jnp.dot(recip, p8_r[...], preferred_element_type=jnp.float32)
    x2 = h * rep + b_r[...]
    x2 = jnp.where(x2 > 0, x2, jnp.expm1(x2))
    rows = pl.program_id(0) * BT + lax.broadcasted_iota(jnp.int32, (BT, 1), 0)
    x2 = jnp.where(rows < N, x2, 0.0)
    he_r[...] = jnp.dot(x2, we_r[...], preferred_element_type=jnp.float32)
    ad_r[...] = jnp.dot(x2, wa_r[...], preferred_element_type=jnp.float32)


def _combine_prep(parts, b, p8, wext, wad):
    width_in = parts.shape[2]
    width = wext.shape[1]
    return pl.pallas_call(
        _combine_prep_body,
        grid=(N_PAD // BT,),
        in_specs=[
            pl.BlockSpec((2, BT, width_in), lambda i: (0, i, 0)),
            pl.BlockSpec((1, HIDDEN), lambda i: (0, 0)),
            pl.BlockSpec((HEADS, HIDDEN), lambda i: (0, 0)),
            pl.BlockSpec((HIDDEN, width), lambda i: (0, 0)),
            pl.BlockSpec((HIDDEN, 16), lambda i: (0, 0)),
        ],
        out_specs=[
            pl.BlockSpec((BT, width), lambda i: (i, 0)),
            pl.BlockSpec((BT, 16), lambda i: (i, 0)),
        ],
        out_shape=[
            jax.ShapeDtypeStruct((N_PAD, width), jnp.float32),
            jax.ShapeDtypeStruct((N_PAD, 16), jnp.float32),
        ],
    )(parts, b, p8, wext, wad)


def _final_body(parts_r, b_r, out_r):
    p = parts_r[0] + parts_r[1]
    v = p[:, :N_CLASSES]
    den = p[:, N_CLASSES:N_CLASSES + 1]
    logits = v / (den + 1e-16) + b_r[...]
    hh = jnp.where(logits > 0, logits, jnp.expm1(logits))
    m = jnp.max(hh, axis=1, keepdims=True)
    out_r[...] = hh - m - jnp.log(
        jnp.sum(jnp.exp(hh - m), axis=1, keepdims=True))


def _final(parts, b):
    width_in = parts.shape[2]
    return pl.pallas_call(
        _final_body,
        grid=(N_PAD // BT,),
        in_specs=[
            pl.BlockSpec((2, BT, width_in), lambda i: (0, i, 0)),
            pl.BlockSpec((1, N_CLASSES), lambda i: (0, 0)),
        ],
        out_specs=pl.BlockSpec((BT, N_CLASSES), lambda i: (i, 0)),
        out_shape=jax.ShapeDtypeStruct((N_PAD, N_CLASSES), jnp.float32),
    )(parts, b)


def _att_matrix(att, heads, ch):
    a = att.reshape(heads, ch)
    m = jnp.eye(heads, dtype=a.dtype)[:, None, :] * a[:, :, None]
    return m.reshape(heads * ch, heads)


def _massage(w, a_s, a_d, heads, ch):
    hoff = heads * ch
    width = hoff + 16
    ms = _att_matrix(a_s, heads, ch)
    md = _att_matrix(a_d, heads, ch)
    din = w.shape[0]
    wext = jnp.concatenate(
        [w, w @ ms, jnp.zeros((din, width - hoff - heads), w.dtype)], axis=1)
    wad = jnp.concatenate(
        [w @ md, jnp.zeros((din, 16 - heads), w.dtype)], axis=1)
    return wext, wad


def kernel(x, edge_index, W1, as1, ad1, b1, W2, as2, ad2, b2,
           W3, as3, ad3, b3):
    ei = edge_index.astype(jnp.int32)
    loop = jnp.arange(N, dtype=jnp.int32)
    padv = jnp.full((E_PAD - E_RAW,), N, jnp.int32)
    src = jnp.concatenate([ei[0], loop, padv])
    dst = jnp.concatenate([ei[1], loop, padv])

    w1e, w1d = _massage(W1, as1, ad1, HEADS, PER_HEAD)
    w2e, w2d = _massage(W2, as2, ad2, HEADS, PER_HEAD)
    w3e, w3d = _massage(W3, as3, ad3, 1, N_CLASSES)

    p8 = jnp.kron(jnp.eye(HEADS, dtype=jnp.float32),
                  jnp.ones((1, PER_HEAD), jnp.float32))

    xp = jnp.zeros((N_PAD, F_IN), jnp.float32).at[:N].set(x)

    sc_big = _make_sc_edge(HIDDEN + 16, HEADS)
    sc_small = _make_sc_edge(N_CLASSES + 16, 1)

    he1, ad1t = _prep(xp, w1e, w1d)
    parts1 = sc_big(src, dst, he1, ad1t)
    he2, ad2t = _combine_prep(parts1, b1.reshape(1, HIDDEN), p8, w2e, w2d)
    parts2 = sc_big(src, dst, he2, ad2t)
    he3, ad3t = _combine_prep(parts2, b2.reshape(1, HIDDEN), p8, w3e, w3d)
    parts3 = sc_small(src, dst, he3, ad3t)
    out = _final(parts3, b3.reshape(1, N_CLASSES))
    return out[:N]


# speedup vs baseline: 59.1158x; 59.1158x over previous
import functools

import jax
import jax.numpy as jnp
from jax import lax
from jax.experimental import pallas as pl
from jax.experimental.pallas import tpu as pltpu
from jax.experimental.pallas import tpu_sc as plsc

N = 10000
F_IN = 128
HEADS = 8
PER_HEAD = 16
N_CLASSES = 16
HIDDEN = HEADS * PER_HEAD

N_PAD = 10240
E_RAW = 320000 + N
NTILES = 32
CHUNK = 128
EPT = 10368
E_PAD = NTILES * EPT
BT = 256


def _sc_edge_body(width, heads, nsc,
                  src_r, dst_r, hx_r, ad_r, out_r,
                  sidx, didx, hbuf, adbuf, acc, sem1, sem2):
    hoff = heads * PER_HEAD
    c = lax.axis_index("c")
    s = lax.axis_index("s")
    wid = s * nsc + c
    nvec = width // 16
    rpt = N_PAD // 16

    def zrow(e, _):
        for v in range(nvec):
            hbuf[e, pl.ds(16 * v, 16)] = jnp.zeros((16,), jnp.float32)
        return 0
    lax.fori_loop(0, CHUNK, zrow, 0)
    for k in range(rpt // CHUNK):
        pltpu.sync_copy(hbuf, acc.at[pl.ds(s * rpt + k * CHUNK, CHUNK)])
    plsc.subcore_barrier()

    lane = lax.iota(jnp.int32, 16)
    lmask = lane < heads

    def chunk_body(i, _):
        base = wid * EPT + i * CHUNK
        pltpu.sync_copy(src_r.at[pl.ds(base, CHUNK)], sidx)
        pltpu.sync_copy(dst_r.at[pl.ds(base, CHUNK)], didx)
        cp1 = pltpu.async_copy(hx_r.at[sidx], hbuf, sem1)
        cp2 = pltpu.async_copy(ad_r.at[didx], adbuf, sem2)
        cp1.wait()
        cp2.wait()

        def edge(e, _):
            asv = hbuf[e, pl.ds(hoff, 16)]
            adv = adbuf[e, :]
            z = asv + adv
            lr = jnp.maximum(z, 0.2 * z)
            ev = jnp.where(lmask, jnp.exp(lr), 0.0)
            hbuf[e, pl.ds(hoff, 16)] = ev
            for j in range(heads):
                hv = hbuf[e, pl.ds(16 * j, 16)]
                hbuf[e, pl.ds(16 * j, 16)] = ev[j] * hv
            return 0
        lax.fori_loop(0, CHUNK, edge, 0)

        pltpu.sync_copy(hbuf, acc.at[didx], add=True)
        return 0
    lax.fori_loop(0, EPT // CHUNK, chunk_body, 0)

    plsc.subcore_barrier()
    pltpu.sync_copy(acc.at[pl.ds(s * rpt, rpt)],
                    out_r.at[c].at[pl.ds(s * rpt, rpt)])


def _make_sc_edge(width, heads):
    info = plsc.get_sparse_core_info()
    nsc = info.num_cores
    mesh = plsc.VectorSubcoreMesh(core_axis_name="c", subcore_axis_name="s")
    return functools.partial(
        pl.kernel,
        out_type=jax.ShapeDtypeStruct((nsc, N_PAD, width), jnp.float32),
        mesh=mesh,
        compiler_params=pltpu.CompilerParams(use_tc_tiling_on_sc=False),
        scratch_types=[
            pltpu.VMEM((CHUNK,), jnp.int32),
            pltpu.VMEM((CHUNK,), jnp.int32),
            pltpu.VMEM((CHUNK, width), jnp.float32),
            pltpu.VMEM((CHUNK, 16), jnp.float32),
            pltpu.VMEM_SHARED((N_PAD, width), jnp.float32),
            pltpu.SemaphoreType.DMA,
            pltpu.SemaphoreType.DMA,
        ],
    )(functools.partial(_sc_edge_body, width, heads, nsc))


def _prep_body(x_r, we_r, wa_r, he_r, ad_r):
    x = x_r[...]
    he_r[...] = jnp.dot(x, we_r[...], preferred_element_type=jnp.float32)
    ad_r[...] = jnp.dot(x, wa_r[...], preferred_element_type=jnp.float32)


def _prep(xp, wext, wad):
    width = wext.shape[1]
    return pl.pallas_call(
        _prep_body,
        grid=(N_PAD // BT,),
        in_specs=[
            pl.BlockSpec((BT, F_IN), lambda i: (i, 0)),
            pl.BlockSpec((F_IN, width), lambda i: (0, 0)),
            pl.BlockSpec((F_IN, 16), lambda i: (0, 0)),
        ],
        out_specs=[
            pl.BlockSpec((BT, width), lambda i: (i, 0)),
            pl.BlockSpec((BT, 16), lambda i: (i, 0)),
        ],
        out_shape=[
            jax.ShapeDtypeStruct((N_PAD, width), jnp.float32),
            jax.ShapeDtypeStruct((N_PAD, 16), jnp.float32),
        ],
    )(xp, wext, wad)


def _combine_prep_body(parts_r, b_r, p8_r, we_r, wa_r, he_r, ad_r):
    p = parts_r[0] + parts_r[1]
    h = p[:, :HIDDEN]
    den = p[:, HIDDEN:HIDDEN + HEADS]
    recip = 1.0 / (den + 1e-16)
    rep = jnp.dot(recip, p8_r[...], preferred_element_type=jnp.float32)
    x2 = h * rep + b_r[...]
    x2 = jnp.where(x2 > 0, x2, jnp.exp(x2) - 1.0)
    rows = pl.program_id(0) * BT + lax.broadcasted_iota(jnp.int32, (BT, 1), 0)
    x2 = jnp.where(rows < N, x2, 0.0)
    he_r[...] = jnp.dot(x2, we_r[...], preferred_element_type=jnp.float32)
    ad_r[...] = jnp.dot(x2, wa_r[...], preferred_element_type=jnp.float32)


def _combine_prep(parts, b, p8, wext, wad):
    width_in = parts.shape[2]
    width = wext.shape[1]
    return pl.pallas_call(
        _combine_prep_body,
        grid=(N_PAD // BT,),
        in_specs=[
            pl.BlockSpec((2, BT, width_in), lambda i: (0, i, 0)),
            pl.BlockSpec((1, HIDDEN), lambda i: (0, 0)),
            pl.BlockSpec((HEADS, HIDDEN), lambda i: (0, 0)),
            pl.BlockSpec((HIDDEN, width), lambda i: (0, 0)),
            pl.BlockSpec((HIDDEN, 16), lambda i: (0, 0)),
        ],
        out_specs=[
            pl.BlockSpec((BT, width), lambda i: (i, 0)),
            pl.BlockSpec((BT, 16), lambda i: (i, 0)),
        ],
        out_shape=[
            jax.ShapeDtypeStruct((N_PAD, width), jnp.float32),
            jax.ShapeDtypeStruct((N_PAD, 16), jnp.float32),
        ],
    )(parts, b, p8, wext, wad)


def _final_body(parts_r, b_r, out_r):
    p = parts_r[0] + parts_r[1]
    v = p[:, :N_CLASSES]
    den = p[:, N_CLASSES:N_CLASSES + 1]
    logits = v / (den + 1e-16) + b_r[...]
    hh = jnp.where(logits > 0, logits, jnp.exp(logits) - 1.0)
    m = jnp.max(hh, axis=1, keepdims=True)
    out_r[...] = hh - m - jnp.log(
        jnp.sum(jnp.exp(hh - m), axis=1, keepdims=True))


def _final(parts, b):
    width_in = parts.shape[2]
    return pl.pallas_call(
        _final_body,
        grid=(N_PAD // BT,),
        in_specs=[
            pl.BlockSpec((2, BT, width_in), lambda i: (0, i, 0)),
            pl.BlockSpec((1, N_CLASSES), lambda i: (0, 0)),
        ],
        out_specs=pl.BlockSpec((BT, N_CLASSES), lambda i: (i, 0)),
        out_shape=jax.ShapeDtypeStruct((N_PAD, N_CLASSES), jnp.float32),
    )(parts, b)


def _att_matrix(att, heads, ch):
    a = att.reshape(heads, ch)
    m = jnp.eye(heads, dtype=a.dtype)[:, None, :] * a[:, :, None]
    return m.reshape(heads * ch, heads)


def _massage(w, a_s, a_d, heads, ch):
    hoff = heads * ch
    width = hoff + 16
    ms = _att_matrix(a_s, heads, ch)
    md = _att_matrix(a_d, heads, ch)
    din = w.shape[0]
    wext = jnp.concatenate(
        [w, w @ ms, jnp.zeros((din, width - hoff - heads), w.dtype)], axis=1)
    wad = jnp.concatenate(
        [w @ md, jnp.zeros((din, 16 - heads), w.dtype)], axis=1)
    return wext, wad


def kernel(x, edge_index, W1, as1, ad1, b1, W2, as2, ad2, b2,
           W3, as3, ad3, b3):
    ei = edge_index.astype(jnp.int32)
    loop = jnp.arange(N, dtype=jnp.int32)
    padv = jnp.full((E_PAD - E_RAW,), N, jnp.int32)
    src = jnp.concatenate([ei[0], loop, padv])
    dst = jnp.concatenate([ei[1], loop, padv])

    w1e, w1d = _massage(W1, as1, ad1, HEADS, PER_HEAD)
    w2e, w2d = _massage(W2, as2, ad2, HEADS, PER_HEAD)
    w3e, w3d = _massage(W3, as3, ad3, 1, N_CLASSES)

    p8 = jnp.kron(jnp.eye(HEADS, dtype=jnp.float32),
                  jnp.ones((1, PER_HEAD), jnp.float32))

    xp = jnp.zeros((N_PAD, F_IN), jnp.float32).at[:N].set(x)

    sc_big = _make_sc_edge(HIDDEN + 16, HEADS)
    sc_small = _make_sc_edge(N_CLASSES + 16, 1)

    he1, ad1t = _prep(xp, w1e, w1d)
    parts1 = sc_big(src, dst, he1, ad1t)
    he2, ad2t = _combine_prep(parts1, b1.reshape(1, HIDDEN), p8, w2e, w2d)
    parts2 = sc_big(src, dst, he2, ad2t)
    he3, ad3t = _combine_prep(parts2, b2.reshape(1, HIDDEN), p8, w3e, w3d)
    parts3 = sc_small(src, dst, he3, ad3t)
    out = _final(parts3, b3.reshape(1, N_CLASSES))
    return out[:N]


# speedup vs baseline: 108.7047x; 1.8388x over previous
import functools

import jax
import jax.numpy as jnp
from jax import lax
from jax.experimental import pallas as pl
from jax.experimental.pallas import tpu as pltpu
from jax.experimental.pallas import tpu_sc as plsc

N = 10000
F_IN = 128
HEADS = 8
PER_HEAD = 16
N_CLASSES = 16
HIDDEN = HEADS * PER_HEAD

N_PAD = 10112
E_RAW = 320000 + N
NTILES = 32
CHUNK = 80
EPT = 10320
E_PAD = NTILES * EPT
BT = 128


NBUF = 3


def _sc_edge_body(width, heads, nsc,
                  src_r, dst_r, hx_r, ad_r, out_r, *scratch):
    hoff = heads * PER_HEAD
    sis = scratch[0:3]
    dis = scratch[3:6]
    hbs = scratch[6:9]
    abs_ = scratch[9:12]
    acc = scratch[12]
    ghs = scratch[13:16]
    gas = scratch[16:19]
    sss = scratch[19:22]

    c = lax.axis_index("c")
    s = lax.axis_index("s")
    wid = s * nsc + c
    ebase = wid * EPT
    nvec = width // 16
    rpt = N_PAD // 16
    nch = EPT // CHUNK

    def load_idx(slot, ci):
        base = ebase + ci * CHUNK
        pltpu.sync_copy(src_r.at[pl.ds(base, CHUNK)], sis[slot])
        pltpu.sync_copy(dst_r.at[pl.ds(base, CHUNK)], dis[slot])

    def start_gather(slot):
        pltpu.async_copy(hx_r.at[sis[slot]], hbs[slot], ghs[slot])
        pltpu.async_copy(ad_r.at[dis[slot]], abs_[slot], gas[slot])

    def wait_gather(slot):
        pltpu.make_async_copy(hx_r.at[sis[slot]], hbs[slot], ghs[slot]).wait()
        pltpu.make_async_copy(ad_r.at[dis[slot]], abs_[slot], gas[slot]).wait()

    def start_scatter(slot):
        pltpu.async_copy(hbs[slot], acc.at[dis[slot]], sss[slot], add=True)

    def wait_scatter(slot):
        pltpu.make_async_copy(hbs[slot], acc.at[dis[slot]], sss[slot]).wait()

    load_idx(0, 0)
    start_gather(0)
    load_idx(1, 1)
    start_gather(1)

    zbuf = hbs[2]

    def zrow(e, _):
        for v in range(nvec):
            zbuf[e, pl.ds(16 * v, 16)] = jnp.zeros((16,), jnp.float32)
        return 0
    lax.fori_loop(0, CHUNK, zrow, 0)
    for k in range(rpt // CHUNK):
        pltpu.sync_copy(zbuf, acc.at[pl.ds(s * rpt + k * CHUNK, CHUNK)])
    rem = rpt % CHUNK
    if rem:
        pltpu.sync_copy(
            zbuf.at[pl.ds(0, rem)],
            acc.at[pl.ds(s * rpt + (rpt // CHUNK) * CHUNK, rem)])
    plsc.subcore_barrier()

    lane = lax.iota(jnp.int32, 16)
    lmask = lane < heads

    def compute(hbuf, adbuf):
        @plsc.parallel_loop(0, CHUNK, unroll=4)
        def edge(e):
            asv = hbuf[e, pl.ds(hoff, 16)]
            adv = adbuf[e, :]
            z = asv + adv
            lr = jnp.maximum(z, 0.2 * z)
            ev = jnp.where(lmask, jnp.exp(lr), 0.0)
            hbuf[e, pl.ds(hoff, 16)] = ev
            for j in range(heads):
                hv = hbuf[e, pl.ds(16 * j, 16)]
                hbuf[e, pl.ds(16 * j, 16)] = ev[j] * hv

    def group(g, _):
        for b in range(NBUF):
            ci = NBUF * g + b
            wait_gather(b)
            compute(hbs[b], abs_[b])
            start_scatter(b)
            b2 = (b + 2) % NBUF
            ci2 = ci + 2

            @pl.when(ci2 < nch)
            def _():
                @pl.when(ci2 >= NBUF)
                def _():
                    wait_scatter(b2)
                load_idx(b2, ci2)
                start_gather(b2)
        return 0
    lax.fori_loop(0, nch // NBUF, group, 0)

    for b in range(NBUF):
        wait_scatter(b)
    plsc.subcore_barrier()
    pltpu.sync_copy(acc.at[pl.ds(s * rpt, rpt)],
                    out_r.at[c].at[pl.ds(s * rpt, rpt)])


def _make_sc_edge(width, heads):
    info = plsc.get_sparse_core_info()
    nsc = info.num_cores
    mesh = plsc.VectorSubcoreMesh(core_axis_name="c", subcore_axis_name="s")
    return functools.partial(
        pl.kernel,
        out_type=jax.ShapeDtypeStruct((nsc, N_PAD, width), jnp.float32),
        mesh=mesh,
        compiler_params=pltpu.CompilerParams(use_tc_tiling_on_sc=False),
        scratch_types=(
            [pltpu.VMEM((CHUNK,), jnp.int32) for _ in range(2 * NBUF)]
            + [pltpu.VMEM((CHUNK, width), jnp.float32) for _ in range(NBUF)]
            + [pltpu.VMEM((CHUNK, 16), jnp.float32) for _ in range(NBUF)]
            + [pltpu.VMEM_SHARED((N_PAD, width), jnp.float32)]
            + [pltpu.SemaphoreType.DMA for _ in range(3 * NBUF)]
        ),
    )(functools.partial(_sc_edge_body, width, heads, nsc))


def _prep_body(x_r, we_r, wa_r, he_r, ad_r):
    x = x_r[...]
    he_r[...] = jnp.dot(x, we_r[...], preferred_element_type=jnp.float32)
    ad_r[...] = jnp.dot(x, wa_r[...], preferred_element_type=jnp.float32)


def _prep(xp, wext, wad):
    width = wext.shape[1]
    return pl.pallas_call(
        _prep_body,
        grid=(N_PAD // BT,),
        in_specs=[
            pl.BlockSpec((BT, F_IN), lambda i: (i, 0)),
            pl.BlockSpec((F_IN, width), lambda i: (0, 0)),
            pl.BlockSpec((F_IN, 16), lambda i: (0, 0)),
        ],
        out_specs=[
            pl.BlockSpec((BT, width), lambda i: (i, 0)),
            pl.BlockSpec((BT, 16), lambda i: (i, 0)),
        ],
        out_shape=[
            jax.ShapeDtypeStruct((N_PAD, width), jnp.float32),
            jax.ShapeDtypeStruct((N_PAD, 16), jnp.float32),
        ],
    )(xp, wext, wad)


def _combine_prep_body(parts_r, b_r, p8_r, we_r, wa_r, he_r, ad_r):
    p = parts_r[0] + parts_r[1]
    h = p[:, :HIDDEN]
    den = p[:, HIDDEN:HIDDEN + HEADS]
    recip = 1.0 / (den + 1e-16)
    rep = jnp.dot(recip, p8_r[...], preferred_element_type=jnp.float32)
    x2 = h * rep + b_r[...]
    x2 = jnp.where(x2 > 0, x2, jnp.exp(x2) - 1.0)
    rows = pl.program_id(0) * BT + lax.broadcasted_iota(jnp.int32, (BT, 1), 0)
    x2 = jnp.where(rows < N, x2, 0.0)
    he_r[...] = jnp.dot(x2, we_r[...], preferred_element_type=jnp.float32)
    ad_r[...] = jnp.dot(x2, wa_r[...], preferred_element_type=jnp.float32)


def _combine_prep(parts, b, p8, wext, wad):
    width_in = parts.shape[2]
    width = wext.shape[1]
    return pl.pallas_call(
        _combine_prep_body,
        grid=(N_PAD // BT,),
        in_specs=[
            pl.BlockSpec((2, BT, width_in), lambda i: (0, i, 0)),
            pl.BlockSpec((1, HIDDEN), lambda i: (0, 0)),
            pl.BlockSpec((HEADS, HIDDEN), lambda i: (0, 0)),
            pl.BlockSpec((HIDDEN, width), lambda i: (0, 0)),
            pl.BlockSpec((HIDDEN, 16), lambda i: (0, 0)),
        ],
        out_specs=[
            pl.BlockSpec((BT, width), lambda i: (i, 0)),
            pl.BlockSpec((BT, 16), lambda i: (i, 0)),
        ],
        out_shape=[
            jax.ShapeDtypeStruct((N_PAD, width), jnp.float32),
            jax.ShapeDtypeStruct((N_PAD, 16), jnp.float32),
        ],
    )(parts, b, p8, wext, wad)


def _final_body(parts_r, b_r, out_r):
    p = parts_r[0] + parts_r[1]
    v = p[:, :N_CLASSES]
    den = p[:, N_CLASSES:N_CLASSES + 1]
    logits = v / (den + 1e-16) + b_r[...]
    hh = jnp.where(logits > 0, logits, jnp.exp(logits) - 1.0)
    m = jnp.max(hh, axis=1, keepdims=True)
    out_r[...] = hh - m - jnp.log(
        jnp.sum(jnp.exp(hh - m), axis=1, keepdims=True))


def _final(parts, b):
    width_in = parts.shape[2]
    return pl.pallas_call(
        _final_body,
        grid=(N_PAD // BT,),
        in_specs=[
            pl.BlockSpec((2, BT, width_in), lambda i: (0, i, 0)),
            pl.BlockSpec((1, N_CLASSES), lambda i: (0, 0)),
        ],
        out_specs=pl.BlockSpec((BT, N_CLASSES), lambda i: (i, 0)),
        out_shape=jax.ShapeDtypeStruct((N_PAD, N_CLASSES), jnp.float32),
    )(parts, b)


def _att_matrix(att, heads, ch):
    a = att.reshape(heads, ch)
    m = jnp.eye(heads, dtype=a.dtype)[:, None, :] * a[:, :, None]
    return m.reshape(heads * ch, heads)


def _massage(w, a_s, a_d, heads, ch):
    hoff = heads * ch
    width = hoff + 16
    ms = _att_matrix(a_s, heads, ch)
    md = _att_matrix(a_d, heads, ch)
    din = w.shape[0]
    wext = jnp.concatenate(
        [w, w @ ms, jnp.zeros((din, width - hoff - heads), w.dtype)], axis=1)
    wad = jnp.concatenate(
        [w @ md, jnp.zeros((din, 16 - heads), w.dtype)], axis=1)
    return wext, wad


def kernel(x, edge_index, W1, as1, ad1, b1, W2, as2, ad2, b2,
           W3, as3, ad3, b3):
    ei = edge_index.astype(jnp.int32)
    loop = jnp.arange(N, dtype=jnp.int32)
    padv = jnp.full((E_PAD - E_RAW,), N, jnp.int32)
    src = jnp.concatenate([ei[0], loop, padv])
    dst = jnp.concatenate([ei[1], loop, padv])

    w1e, w1d = _massage(W1, as1, ad1, HEADS, PER_HEAD)
    w2e, w2d = _massage(W2, as2, ad2, HEADS, PER_HEAD)
    w3e, w3d = _massage(W3, as3, ad3, 1, N_CLASSES)

    p8 = jnp.kron(jnp.eye(HEADS, dtype=jnp.float32),
                  jnp.ones((1, PER_HEAD), jnp.float32))

    xp = jnp.zeros((N_PAD, F_IN), jnp.float32).at[:N].set(x)

    sc_big = _make_sc_edge(HIDDEN + 16, HEADS)
    sc_small = _make_sc_edge(N_CLASSES + 16, 1)

    he1, ad1t = _prep(xp, w1e, w1d)
    parts1 = sc_big(src, dst, he1, ad1t)
    he2, ad2t = _combine_prep(parts1, b1.reshape(1, HIDDEN), p8, w2e, w2d)
    parts2 = sc_big(src, dst, he2, ad2t)
    he3, ad3t = _combine_prep(parts2, b2.reshape(1, HIDDEN), p8, w3e, w3d)
    parts3 = sc_small(src, dst, he3, ad3t)
    out = _final(parts3, b3.reshape(1, N_CLASSES))
    return out[:N]
